# speedup vs baseline: 2.1573x; 2.1573x over previous
import jax
import jax.numpy as jnp
from jax.experimental import pallas as pl


def _add_kernel(x_ref, pe_ref, o_ref):
    o_ref[...] = x_ref[...] + pe_ref[...]


def kernel(x, pe_weight):
    B, S, D = x.shape
    SB = 256
    return pl.pallas_call(
        _add_kernel,
        grid=(S // SB,),
        in_specs=[
            pl.BlockSpec((B, SB, D), lambda s: (0, s, 0)),
            pl.BlockSpec((SB, D), lambda s: (s, 0)),
        ],
        out_specs=pl.BlockSpec((B, SB, D), lambda s: (0, s, 0)),
        out_shape=jax.ShapeDtypeStruct((B, S, D), x.dtype),
    )(x, pe_weight)
